# baseline (speedup 1.0000x reference)
import jax
import jax.numpy as jnp
from jax import lax
from jax.experimental import pallas as pl
from jax.experimental.pallas import tpu as pltpu
from jax.experimental.pallas import tpu_sc as plsc

_N = 2048
_K = 16
_RB = 1024
_PAD = 16
_NW = 32
_CH = 2048


def _topk_body(xt_ref, c_ref, idx_ref):
    b = pl.program_id(0)
    xt = xt_ref[0]
    c = c_ref[0]
    xr = xt[0:1, :]
    yr = xt[1:2, :]
    zr = xt[2:3, :]
    cx = c[:, 0:1]
    cy = c[:, 1:2]
    cz = c[:, 2:3]
    d = (cx - xr) ** 2 + (cy - yr) ** 2 + (cz - zr) ** 2
    q = _N // 2
    iotaf = lax.broadcasted_iota(jnp.int32, (_RB, q), 1).astype(jnp.float32)
    s0, s1 = d[:, :q], d[:, q:]
    i0, i1 = iotaf, iotaf + jnp.float32(q)
    swap = s1 < s0
    s0, s1 = jnp.where(swap, s1, s0), jnp.where(swap, s0, s1)
    i0, i1 = jnp.where(swap, i1, i0), jnp.where(swap, i0, i1)

    jlane = lax.broadcasted_iota(jnp.int32, (_RB, _K), 1)
    big_c = jnp.float32(2 * _N)
    inf = jnp.float32(jnp.inf)
    out = jnp.zeros((_RB, _K), jnp.float32)
    for j in range(_K):
        m = jnp.min(s0, axis=1, keepdims=True)
        cand = jnp.where(s0 == m, i0, big_c)
        amin = jnp.min(cand, axis=1, keepdims=True)
        win = i0 == amin
        out = jnp.where(jlane == j, amin, out)
        s0 = jnp.where(win, s1, s0)
        i0 = jnp.where(win, i1, i0)
        s1 = jnp.where(win, inf, s1)
    idx_ref[0] = out.astype(jnp.int32) + b * _N


def _sc_gather(table_hbm, idx_hbm, out_hbm, idx_v, rows_v, sem):
    wid = lax.axis_index("s") * 2 + lax.axis_index("c")
    rows_per_w = (8 * _N * _K) // _NW
    for ci in range(rows_per_w // _CH):
        base = pl.multiple_of(wid * rows_per_w + ci * _CH, _CH)
        pltpu.sync_copy(
            idx_hbm.at[pl.ds(pl.multiple_of(base // 128, _CH // 128),
                             _CH // 128)],
            idx_v)
        copies = []
        for j in range(_CH // 128):
            copies.append(pltpu.async_copy(
                table_hbm.at[idx_v.at[j]],
                rows_v.at[pl.ds(j * 128, 128)], sem))
        for cp in copies:
            cp.wait()
        pltpu.sync_copy(rows_v, out_hbm.at[pl.ds(base, _CH)])


def _sub_body(g_ref, cp_ref, o_ref):
    cp = cp_ref[...]
    o_ref[...] = g_ref[...] - jnp.tile(cp, (1, _K))


def kernel(pcd):
    b, n, _ = pcd.shape
    xt = jnp.transpose(pcd, (0, 2, 1))
    idx = pl.pallas_call(
        _topk_body,
        grid=(b, n // _RB),
        in_specs=[
            pl.BlockSpec((1, 3, _N), lambda i, r: (i, 0, 0)),
            pl.BlockSpec((1, _RB, 3), lambda i, r: (i, r, 0)),
        ],
        out_specs=pl.BlockSpec((1, _RB, _K), lambda i, r: (i, r, 0)),
        out_shape=jax.ShapeDtypeStruct((b, n, _K), jnp.int32),
    )(xt, pcd)

    table = jnp.pad(pcd.reshape(b * n, 3), ((0, 0), (0, _PAD - 3)))
    idx2d = idx.reshape(b * n * _K // 128, 128)

    mesh = plsc.VectorSubcoreMesh(core_axis_name="c", subcore_axis_name="s")
    gathered = pl.kernel(
        _sc_gather,
        mesh=mesh,
        out_type=jax.ShapeDtypeStruct((b * n * _K, _PAD), jnp.float32),
        scratch_types=[
            pltpu.VMEM((_CH // 128, 128), jnp.int32),
            pltpu.VMEM((_CH, _PAD), jnp.float32),
            pltpu.SemaphoreType.DMA,
        ],
        compiler_params=pltpu.CompilerParams(use_tc_tiling_on_sc=False),
    )(table, idx2d)

    nb = pl.pallas_call(
        _sub_body,
        grid=(b * n // 1024,),
        in_specs=[
            pl.BlockSpec((1024, _K * _PAD), lambda i: (i, 0)),
            pl.BlockSpec((1024, _PAD), lambda i: (i, 0)),
        ],
        out_specs=pl.BlockSpec((1024, _K * _PAD), lambda i: (i, 0)),
        out_shape=jax.ShapeDtypeStruct((b * n, _K * _PAD), jnp.float32),
    )(gathered.reshape(b * n, _K * _PAD), table)

    neighborhood = nb.reshape(b, n, _K, _PAD)[..., :3]
    return neighborhood, pcd


# speedup vs baseline: 1.0027x; 1.0027x over previous
import jax
import jax.numpy as jnp
from jax import lax
from jax.experimental import pallas as pl
from jax.experimental.pallas import tpu as pltpu
from jax.experimental.pallas import tpu_sc as plsc

_N = 2048
_K = 16
_RB = 512
_PAD = 16
_NW = 32
_CH = 2048


def _topk_body(xt_ref, c_ref, idx_ref):
    b = pl.program_id(0)
    xt = xt_ref[0]
    c = c_ref[0]
    xr = xt[0:1, :]
    yr = xt[1:2, :]
    zr = xt[2:3, :]
    cx = c[:, 0:1]
    cy = c[:, 1:2]
    cz = c[:, 2:3]
    d = (cx - xr) ** 2 + (cy - yr) ** 2 + (cz - zr) ** 2
    q = _N // 2
    iotaf = lax.broadcasted_iota(jnp.int32, (_RB, q), 1).astype(jnp.float32)
    s0, s1 = d[:, :q], d[:, q:]
    i0, i1 = iotaf, iotaf + jnp.float32(q)
    swap = s1 < s0
    s0, s1 = jnp.where(swap, s1, s0), jnp.where(swap, s0, s1)
    i0, i1 = jnp.where(swap, i1, i0), jnp.where(swap, i0, i1)

    jlane = lax.broadcasted_iota(jnp.int32, (_RB, _K), 1)
    big_c = jnp.float32(2 * _N)
    inf = jnp.float32(jnp.inf)
    out = jnp.zeros((_RB, _K), jnp.float32)
    for j in range(_K):
        m = jnp.min(s0, axis=1, keepdims=True)
        cand = jnp.where(s0 == m, i0, big_c)
        amin = jnp.min(cand, axis=1, keepdims=True)
        win = i0 == amin
        out = jnp.where(jlane == j, amin, out)
        s0 = jnp.where(win, s1, s0)
        i0 = jnp.where(win, i1, i0)
        s1 = jnp.where(win, inf, s1)
    idx_ref[0] = out.astype(jnp.int32) + b * _N


def _sc_gather(table_hbm, idx_hbm, out_hbm, idx_v, rows_v, sem):
    wid = lax.axis_index("s") * 2 + lax.axis_index("c")
    rows_per_w = (8 * _N * _K) // _NW
    for ci in range(rows_per_w // _CH):
        base = pl.multiple_of(wid * rows_per_w + ci * _CH, _CH)
        pltpu.sync_copy(
            idx_hbm.at[pl.ds(pl.multiple_of(base // 128, _CH // 128),
                             _CH // 128)],
            idx_v)
        copies = []
        for j in range(_CH // 128):
            copies.append(pltpu.async_copy(
                table_hbm.at[idx_v.at[j]],
                rows_v.at[pl.ds(j * 128, 128)], sem))
        for cp in copies:
            cp.wait()
        pltpu.sync_copy(rows_v, out_hbm.at[pl.ds(base, _CH)])


def _sub_body(g_ref, cp_ref, o_ref):
    cp = cp_ref[...]
    o_ref[...] = g_ref[...] - jnp.tile(cp, (1, _K))


def kernel(pcd):
    b, n, _ = pcd.shape
    xt = jnp.transpose(pcd, (0, 2, 1))
    idx = pl.pallas_call(
        _topk_body,
        grid=(b, n // _RB),
        in_specs=[
            pl.BlockSpec((1, 3, _N), lambda i, r: (i, 0, 0)),
            pl.BlockSpec((1, _RB, 3), lambda i, r: (i, r, 0)),
        ],
        out_specs=pl.BlockSpec((1, _RB, _K), lambda i, r: (i, r, 0)),
        out_shape=jax.ShapeDtypeStruct((b, n, _K), jnp.int32),
    )(xt, pcd)

    table = jnp.pad(pcd.reshape(b * n, 3), ((0, 0), (0, _PAD - 3)))
    idx2d = idx.reshape(b * n * _K // 128, 128)

    mesh = plsc.VectorSubcoreMesh(core_axis_name="c", subcore_axis_name="s")
    gathered = pl.kernel(
        _sc_gather,
        mesh=mesh,
        out_type=jax.ShapeDtypeStruct((b * n * _K, _PAD), jnp.float32),
        scratch_types=[
            pltpu.VMEM((_CH // 128, 128), jnp.int32),
            pltpu.VMEM((_CH, _PAD), jnp.float32),
            pltpu.SemaphoreType.DMA,
        ],
        compiler_params=pltpu.CompilerParams(use_tc_tiling_on_sc=False),
    )(table, idx2d)

    nb = pl.pallas_call(
        _sub_body,
        grid=(b * n // 1024,),
        in_specs=[
            pl.BlockSpec((1024, _K * _PAD), lambda i: (i, 0)),
            pl.BlockSpec((1024, _PAD), lambda i: (i, 0)),
        ],
        out_specs=pl.BlockSpec((1024, _K * _PAD), lambda i: (i, 0)),
        out_shape=jax.ShapeDtypeStruct((b * n, _K * _PAD), jnp.float32),
    )(gathered.reshape(b * n, _K * _PAD), table)

    neighborhood = nb.reshape(b, n, _K, _PAD)[..., :3]
    return neighborhood, pcd
